# Initial kernel scaffold; baseline (speedup 1.0000x reference)
import jax
import jax.numpy as jnp
from jax.experimental import pallas as pl


def kernel(x, Wr1, br1, Wdom, Wop, Wtask, Wgate, Wm1, bm1, Wm2, bm2, Wl1, bl1, Wl2, bl2, Wf1, bf1, Wf2, bf2):
    raise NotImplementedError("write your pallas kernel here")


# speedup vs baseline: 1.3123x; 1.3123x over previous
import functools

import jax
import jax.numpy as jnp
from jax.experimental import pallas as pl
from jax.experimental.pallas import tpu as pltpu

B, S, D = 2, 2048, 1024
HR, HE, HF = 256, 4096, 1024

BS = 512
HEB = 1024
NK = HE // HEB


def _router_kernel(x_ref, wr1_ref, br1_ref, wdom_ref, wgate_ref,
                   dlog_ref, glog_ref):
    pooled = jnp.mean(x_ref[...], axis=1)
    h = jnp.tanh(
        jnp.dot(pooled, wr1_ref[...], preferred_element_type=jnp.float32)
        + br1_ref[...])
    dlog_ref[...] = jnp.dot(h, wdom_ref[...],
                            preferred_element_type=jnp.float32)
    glog_ref[...] = jnp.dot(h, wgate_ref[...],
                            preferred_element_type=jnp.float32)


def _expert_kernel(dom_ref, conf_ref, x_ref, w1_ref, b1_ref, w2_ref, b2_ref,
                   wf1_ref, bf1_ref, wf2_ref, bf2_ref, o_ref, acc_ref):
    b = pl.program_id(0)
    k = pl.program_id(2)
    xb = x_ref[0]
    h = jax.nn.gelu(
        jnp.dot(xb, w1_ref[0], preferred_element_type=jnp.float32)
        + b1_ref[0])
    part = jnp.dot(h, w2_ref[0], preferred_element_type=jnp.float32)

    @pl.when(k == 0)
    def _():
        acc_ref[...] = part

    @pl.when(k > 0)
    def _():
        acc_ref[...] += part

    @pl.when(k == NK - 1)
    def _():
        e = acc_ref[...] + b2_ref[0]
        t = jnp.tanh(
            jnp.dot(e, wf1_ref[...], preferred_element_type=jnp.float32)
            + bf1_ref[...])
        f = jnp.dot(t, wf2_ref[...],
                    preferred_element_type=jnp.float32) + bf2_ref[...]
        o_ref[0] = conf_ref[b] * f + xb


@jax.jit
def kernel(x, Wr1, br1, Wdom, Wop, Wtask, Wgate, Wm1, bm1, Wm2, bm2,
           Wl1, bl1, Wl2, bl2, Wf1, bf1, Wf2, bf2):
    del Wop, Wtask

    dlog, glog = pl.pallas_call(
        _router_kernel,
        out_shape=(
            jax.ShapeDtypeStruct((B, 2), jnp.float32),
            jax.ShapeDtypeStruct((B, 2), jnp.float32),
        ),
    )(x, Wr1, br1.reshape(1, HR), Wdom, Wgate)

    dom = (dlog[:, 1] > dlog[:, 0]).astype(jnp.int32)
    gmax = jnp.max(glog, axis=1, keepdims=True)
    eg = jnp.exp(glog - gmax)
    conf = jnp.take_along_axis(eg, dom[:, None], axis=1)[:, 0] / jnp.sum(eg, axis=1)

    W1s = jnp.stack([Wm1, Wl1])
    b1s = jnp.stack([bm1, bl1]).reshape(2, 1, HE)
    W2s = jnp.stack([Wm2, Wl2])
    b2s = jnp.stack([bm2, bl2]).reshape(2, 1, D)

    grid = (B, S // BS, NK)
    out = pl.pallas_call(
        _expert_kernel,
        grid_spec=pltpu.PrefetchScalarGridSpec(
            num_scalar_prefetch=2,
            grid=grid,
            in_specs=[
                pl.BlockSpec((1, BS, D), lambda b, s, k, dom, conf: (b, s, 0)),
                pl.BlockSpec((1, D, HEB),
                             lambda b, s, k, dom, conf: (dom[b], 0, k)),
                pl.BlockSpec((1, 1, HEB),
                             lambda b, s, k, dom, conf: (dom[b], 0, k)),
                pl.BlockSpec((1, HEB, D),
                             lambda b, s, k, dom, conf: (dom[b], k, 0)),
                pl.BlockSpec((1, 1, D),
                             lambda b, s, k, dom, conf: (dom[b], 0, 0)),
                pl.BlockSpec((D, HF), lambda b, s, k, dom, conf: (0, 0)),
                pl.BlockSpec((1, HF), lambda b, s, k, dom, conf: (0, 0)),
                pl.BlockSpec((HF, D), lambda b, s, k, dom, conf: (0, 0)),
                pl.BlockSpec((1, D), lambda b, s, k, dom, conf: (0, 0)),
            ],
            out_specs=pl.BlockSpec((1, BS, D),
                                   lambda b, s, k, dom, conf: (b, s, 0)),
            scratch_shapes=[pltpu.VMEM((BS, D), jnp.float32)],
        ),
        out_shape=jax.ShapeDtypeStruct((B, S, D), jnp.float32),
    )(dom, conf, x, W1s, b1s, W2s, b2s, Wf1, bf1.reshape(1, HF), Wf2,
      bf2.reshape(1, D))
    return out
